# baseline (speedup 1.0000x reference)
import jax
import jax.numpy as jnp
from jax.experimental import pallas as pl

DROP = 0.1
KEEP = 1.0 - DROP
SCALE = 1.0 / KEEP
SCALE2 = SCALE ** 2


ROWS = 512


def _vdp_kernel(mu_ref, k_ref, wrow_ref, wcol_ref, sig_ref, mu_out_ref,
                sig_out_ref):
    mu = mu_ref[0]
    k = k_ref[0]
    mu_out_ref[0] = jnp.where(k != 0.0, mu / KEEP, 0.0)
    wrow = wrow_ref[0]
    wcol = wcol_ref[0]
    sig_out_ref[0] = sig_ref[0] * wrow * wcol


def kernel(mu_in, Sigma_in):
    B, H = mu_in.shape
    keep_mask = jax.random.bernoulli(jax.random.key(42), KEEP, mu_in.shape)
    k = keep_mask.astype(jnp.float32)
    nz = jnp.logical_and(keep_mask, mu_in != 0.0)
    wrow = jnp.where(nz, jnp.float32(SCALE2), 0.0).reshape(B, 1, H)
    wcol = nz.astype(jnp.float32).reshape(B, H, 1)

    mu_out3, Sigma_out = pl.pallas_call(
        _vdp_kernel,
        grid=(B, H // ROWS),
        in_specs=[
            pl.BlockSpec((1, 1, H), lambda b, r: (b, 0, 0)),
            pl.BlockSpec((1, 1, H), lambda b, r: (b, 0, 0)),
            pl.BlockSpec((1, 1, H), lambda b, r: (b, 0, 0)),
            pl.BlockSpec((1, ROWS, 1), lambda b, r: (b, r, 0)),
            pl.BlockSpec((1, ROWS, H), lambda b, r: (b, r, 0)),
        ],
        out_specs=[
            pl.BlockSpec((1, 1, H), lambda b, r: (b, 0, 0)),
            pl.BlockSpec((1, ROWS, H), lambda b, r: (b, r, 0)),
        ],
        out_shape=[
            jax.ShapeDtypeStruct((B, 1, H), jnp.float32),
            jax.ShapeDtypeStruct((B, H, H), jnp.float32),
        ],
    )(mu_in.reshape(B, 1, H), k.reshape(B, 1, H), wrow, wcol, Sigma_in)
    return mu_out3.reshape(B, H), Sigma_out


# speedup vs baseline: 1.0878x; 1.0878x over previous
import jax
import jax.numpy as jnp
from jax.experimental import pallas as pl

DROP = 0.1
KEEP = 1.0 - DROP
SCALE = 1.0 / KEEP
SCALE2 = SCALE ** 2


BB = 2


def _vdp_kernel(mu_ref, k_ref, wrow_ref, wcol_ref, sig_ref, mu_out_ref,
                sig_out_ref):
    mu = mu_ref[...]
    k = k_ref[...]
    mu_out_ref[...] = jnp.where(k != 0.0, mu / KEEP, 0.0)
    wrow = wrow_ref[...]
    wcol = wcol_ref[...]
    sig_out_ref[...] = sig_ref[...] * wrow * wcol


def kernel(mu_in, Sigma_in):
    B, H = mu_in.shape
    keep_mask = jax.random.bernoulli(jax.random.key(42), KEEP, mu_in.shape)
    k = keep_mask.astype(jnp.float32)
    nz = jnp.logical_and(keep_mask, mu_in != 0.0)
    wrow = jnp.where(nz, jnp.float32(SCALE2), 0.0).reshape(B, 1, H)
    wcol = nz.astype(jnp.float32).reshape(B, H, 1)

    mu_out3, Sigma_out = pl.pallas_call(
        _vdp_kernel,
        grid=(B // BB,),
        in_specs=[
            pl.BlockSpec((BB, 1, H), lambda b: (b, 0, 0)),
            pl.BlockSpec((BB, 1, H), lambda b: (b, 0, 0)),
            pl.BlockSpec((BB, 1, H), lambda b: (b, 0, 0)),
            pl.BlockSpec((BB, H, 1), lambda b: (b, 0, 0)),
            pl.BlockSpec((BB, H, H), lambda b: (b, 0, 0)),
        ],
        out_specs=[
            pl.BlockSpec((BB, 1, H), lambda b: (b, 0, 0)),
            pl.BlockSpec((BB, H, H), lambda b: (b, 0, 0)),
        ],
        out_shape=[
            jax.ShapeDtypeStruct((B, 1, H), jnp.float32),
            jax.ShapeDtypeStruct((B, H, H), jnp.float32),
        ],
    )(mu_in.reshape(B, 1, H), k.reshape(B, 1, H), wrow, wcol, Sigma_in)
    return mu_out3.reshape(B, H), Sigma_out


# speedup vs baseline: 1.1024x; 1.0134x over previous
import jax
import jax.numpy as jnp
from jax.experimental import pallas as pl

DROP = 0.1
KEEP = 1.0 - DROP
SCALE = 1.0 / KEEP
SCALE2 = SCALE ** 2

BB = 2


def _vdp_kernel(mu_ref, k_ref, wcol_ref, sig_ref, mu_out_ref, sig_out_ref):
    b = pl.program_id(0)
    mu = mu_ref[...]
    k = k_ref[...]
    mu_out_ref[...] = jnp.where(k != 0.0, mu / KEEP, 0.0)
    mu_s = mu_ref[pl.ds(b * BB, BB)]
    k_s = k_ref[pl.ds(b * BB, BB)]
    wrow = jnp.where((k_s != 0.0) & (mu_s != 0.0), SCALE2, 0.0)
    wcol = wcol_ref[pl.ds(b * BB, BB)]
    sig_out_ref[...] = sig_ref[...] * wrow * wcol


def kernel(mu_in, Sigma_in):
    B, H = mu_in.shape
    keep_mask = jax.random.bernoulli(jax.random.key(42), KEEP, mu_in.shape)
    k = keep_mask.astype(jnp.float32)
    nz = jnp.logical_and(keep_mask, mu_in != 0.0)
    wcol = nz.astype(jnp.float32).reshape(B, H, 1)

    mu_out3, Sigma_out = pl.pallas_call(
        _vdp_kernel,
        grid=(B // BB,),
        in_specs=[
            pl.BlockSpec((B, 1, H), lambda b: (0, 0, 0)),
            pl.BlockSpec((B, 1, H), lambda b: (0, 0, 0)),
            pl.BlockSpec((B, H, 1), lambda b: (0, 0, 0)),
            pl.BlockSpec((BB, H, H), lambda b: (b, 0, 0)),
        ],
        out_specs=[
            pl.BlockSpec((B, 1, H), lambda b: (0, 0, 0)),
            pl.BlockSpec((BB, H, H), lambda b: (b, 0, 0)),
        ],
        out_shape=[
            jax.ShapeDtypeStruct((B, 1, H), jnp.float32),
            jax.ShapeDtypeStruct((B, H, H), jnp.float32),
        ],
    )(mu_in.reshape(B, 1, H), k.reshape(B, 1, H), wcol, Sigma_in)
    return mu_out3.reshape(B, H), Sigma_out


# speedup vs baseline: 1.2726x; 1.1544x over previous
import jax
import jax.numpy as jnp
from jax.experimental import pallas as pl

DROP = 0.1
KEEP = 1.0 - DROP
SCALE = 1.0 / KEEP
SCALE2 = SCALE ** 2

BB = 2


def _vdp_kernel(mu_ref, k_ref, sig_ref, mu_out_ref, sig_out_ref):
    b = pl.program_id(0)
    mu = mu_ref[...]
    k = k_ref[...]
    mu_out_ref[...] = jnp.where(k != 0.0, mu / KEEP, 0.0)
    mu_s = mu_ref[pl.ds(b * BB, BB)]
    k_s = k_ref[pl.ds(b * BB, BB)]
    nz = (k_s != 0.0) & (mu_s != 0.0)
    wrow = jnp.where(nz, SCALE2, 0.0)
    wcol = jnp.swapaxes(jnp.where(nz, 1.0, 0.0), 1, 2)
    sig_out_ref[...] = sig_ref[...] * wrow * wcol


def kernel(mu_in, Sigma_in):
    B, H = mu_in.shape
    keep_mask = jax.random.bernoulli(jax.random.key(42), KEEP, mu_in.shape)
    k = keep_mask.astype(jnp.float32)

    mu_out3, Sigma_out = pl.pallas_call(
        _vdp_kernel,
        grid=(B // BB,),
        in_specs=[
            pl.BlockSpec((B, 1, H), lambda b: (0, 0, 0)),
            pl.BlockSpec((B, 1, H), lambda b: (0, 0, 0)),
            pl.BlockSpec((BB, H, H), lambda b: (b, 0, 0)),
        ],
        out_specs=[
            pl.BlockSpec((B, 1, H), lambda b: (0, 0, 0)),
            pl.BlockSpec((BB, H, H), lambda b: (b, 0, 0)),
        ],
        out_shape=[
            jax.ShapeDtypeStruct((B, 1, H), jnp.float32),
            jax.ShapeDtypeStruct((B, H, H), jnp.float32),
        ],
    )(mu_in.reshape(B, 1, H), k.reshape(B, 1, H), Sigma_in)
    return mu_out3.reshape(B, H), Sigma_out


# speedup vs baseline: 1.2755x; 1.0023x over previous
import jax
import jax.numpy as jnp
from jax.experimental import pallas as pl

DROP = 0.1
KEEP = 1.0 - DROP
SCALE = 1.0 / KEEP
SCALE2 = SCALE ** 2

BB = 2


def _vdp_kernel(mu_ref, k_ref, sig_ref, mu_out_ref, sig_out_ref):
    b = pl.program_id(0)
    mu = mu_ref[...]
    k = k_ref[...]
    mu_out_ref[...] = jnp.where(k != 0.0, mu / KEEP, 0.0)
    mu_s = mu_ref[pl.ds(b * BB, BB)]
    k_s = k_ref[pl.ds(b * BB, BB)]
    nz = (k_s != 0.0) & (mu_s != 0.0)
    wrow = jnp.where(nz, SCALE2, 0.0)
    nzcol = jnp.swapaxes(nz, 1, 2)
    sig_out_ref[...] = jnp.where(nzcol, sig_ref[...] * wrow, 0.0)


def kernel(mu_in, Sigma_in):
    B, H = mu_in.shape
    keep_mask = jax.random.bernoulli(jax.random.key(42), KEEP, mu_in.shape)
    k = keep_mask.astype(jnp.float32)

    mu_out3, Sigma_out = pl.pallas_call(
        _vdp_kernel,
        grid=(B // BB,),
        in_specs=[
            pl.BlockSpec((B, 1, H), lambda b: (0, 0, 0)),
            pl.BlockSpec((B, 1, H), lambda b: (0, 0, 0)),
            pl.BlockSpec((BB, H, H), lambda b: (b, 0, 0)),
        ],
        out_specs=[
            pl.BlockSpec((B, 1, H), lambda b: (0, 0, 0)),
            pl.BlockSpec((BB, H, H), lambda b: (b, 0, 0)),
        ],
        out_shape=[
            jax.ShapeDtypeStruct((B, 1, H), jnp.float32),
            jax.ShapeDtypeStruct((B, H, H), jnp.float32),
        ],
    )(mu_in.reshape(B, 1, H), k.reshape(B, 1, H), Sigma_in)
    return mu_out3.reshape(B, H), Sigma_out
